# Initial kernel scaffold; baseline (speedup 1.0000x reference)
import jax
import jax.numpy as jnp
from jax.experimental import pallas as pl


def kernel(hex_features, vertex_features, edge_features, player_features, current_player, params):
    raise NotImplementedError("write your pallas kernel here")


# speedup vs baseline: 1.0003x; 1.0003x over previous
import numpy as np
import jax
import jax.numpy as jnp
from jax.experimental import pallas as pl

_B = 2048
_HID = 64
_HEADS = 4
_OUT_DIM = 256
_COUNTS = {"hex": 19, "vertex": 54, "edge": 72}
_RELS = [("hex", "h2v", "vertex"), ("vertex", "v2h", "hex"),
         ("vertex", "v2e", "edge"), ("edge", "e2v", "vertex"),
         ("vertex", "v2v", "vertex")]


def _topo():
    rng = np.random.RandomState(0)
    hex_vertices = rng.randint(0, 54, size=(19, 6))
    v0 = rng.randint(0, 54, size=72)
    v1 = (v0 + 1 + rng.randint(0, 53, size=72)) % 54
    edge_vertices = np.stack([v0, v1], axis=1)
    h2v = np.stack([np.repeat(np.arange(19), 6), hex_vertices.reshape(-1)])
    v2h = h2v[::-1].copy()
    e2v = np.stack([np.repeat(np.arange(72), 2), edge_vertices.reshape(-1)])
    v2e = e2v[::-1].copy()
    v2v = np.stack([np.concatenate([edge_vertices[:, 0], edge_vertices[:, 1]]),
                    np.concatenate([edge_vertices[:, 1], edge_vertices[:, 0]])])
    return {"h2v": h2v, "v2h": v2h, "v2e": v2e, "e2v": e2v, "v2v": v2v}


_EI = _topo()


def _gat(x_src, x_dst, src, dst, p, n_dst):
    xs = (x_src @ p["w"]).reshape(-1, _HEADS, _HID)
    xd = (x_dst @ p["w"]).reshape(-1, _HEADS, _HID)
    a = (xs * p["att_src"]).sum(-1)[src] + (xd * p["att_dst"]).sum(-1)[dst]
    a = jax.nn.leaky_relu(a, 0.2)
    amax = jax.ops.segment_max(a, dst, num_segments=n_dst)
    e = jnp.exp(a - amax[dst])
    den = jax.ops.segment_sum(e, dst, num_segments=n_dst)
    alpha = e / (den[dst] + 1e-16)
    out = jax.ops.segment_sum(xs[src] * alpha[:, :, None], dst, num_segments=n_dst)
    return out.mean(axis=1) + p["bias"]


def _ln(x, g, b):
    mu = x.mean(-1, keepdims=True)
    var = ((x - mu) ** 2).mean(-1, keepdims=True)
    return (x - mu) / jnp.sqrt(var + 1e-5) * g + b


def _mlp_body(h_ref, w1_ref, b1_ref, w2_ref, b2_ref, o_ref):
    h = h_ref[...]
    z = jax.nn.relu(h @ w1_ref[...] + b1_ref[...][None, :])
    o_ref[...] = z @ w2_ref[...] + b2_ref[...][None, :]


def kernel(hex_features, vertex_features, edge_features, player_features,
           current_player, params):
    bsz = hex_features.shape[0]
    feats = {"hex": hex_features, "vertex": vertex_features, "edge": edge_features}
    x = {}
    for nt in _COUNTS:
        p = params["proj"][nt]
        x[nt] = feats[nt].reshape(bsz * _COUNTS[nt], -1) @ p["w"] + p["b"]
    bei = {}
    for (s, r, d) in _RELS:
        ei = _EI[r]
        so = np.arange(bsz) * _COUNTS[s]
        do = np.arange(bsz) * _COUNTS[d]
        src = (ei[0][None, :] + so[:, None]).reshape(-1)
        dst = (ei[1][None, :] + do[:, None]).reshape(-1)
        bei[r] = (jnp.asarray(src, jnp.int32), jnp.asarray(dst, jnp.int32))
    for layer in params["layers"]:
        out = {nt: jnp.zeros((bsz * _COUNTS[nt], _HID), jnp.float32) for nt in _COUNTS}
        for (s, r, d) in _RELS:
            src, dst = bei[r]
            out[d] = out[d] + _gat(x[s], x[d], src, dst, layer["convs"][r], bsz * _COUNTS[d])
        x = {nt: jax.nn.relu(_ln(out[nt], layer["norms"][nt]["g"], layer["norms"][nt]["b"]))
             for nt in _COUNTS}
    pools = [x[nt].reshape(bsz, _COUNTS[nt], _HID).mean(axis=1)
             for nt in ["hex", "vertex", "edge"]]
    cur = player_features[jnp.arange(bsz), current_player]
    h = jnp.concatenate(pools + [cur], axis=-1)
    m = params["mlp"]
    out = pl.pallas_call(
        _mlp_body,
        out_shape=jax.ShapeDtypeStruct((bsz, _OUT_DIM), jnp.float32),
    )(h, m["w1"], m["b1"], m["w2"], m["b2"])
    return out


# speedup vs baseline: 16.0496x; 16.0452x over previous
import numpy as np
import jax
import jax.numpy as jnp
from jax.experimental import pallas as pl

_HID = 64
_HEADS = 4
_OUT_DIM = 256
_PLAYER_DIM = 14
_N_PLAYERS = 4
_NL = 3
_IN_DIMS = {"hex": 9, "vertex": 7, "edge": 5}
_COUNTS = {"hex": 19, "vertex": 54, "edge": 72}
_RELS = [("hex", "h2v", "vertex"), ("vertex", "v2h", "hex"),
         ("vertex", "v2e", "edge"), ("edge", "e2v", "vertex"),
         ("vertex", "v2v", "vertex")]
_NTS = ["hex", "vertex", "edge"]


def _topo():
    rng = np.random.RandomState(0)
    hex_vertices = rng.randint(0, 54, size=(19, 6))
    v0 = rng.randint(0, 54, size=72)
    v1 = (v0 + 1 + rng.randint(0, 53, size=72)) % 54
    edge_vertices = np.stack([v0, v1], axis=1)
    h2v = np.stack([np.repeat(np.arange(19), 6), hex_vertices.reshape(-1)])
    v2h = h2v[::-1].copy()
    e2v = np.stack([np.repeat(np.arange(72), 2), edge_vertices.reshape(-1)])
    v2e = e2v[::-1].copy()
    v2v = np.stack([np.concatenate([edge_vertices[:, 0], edge_vertices[:, 1]]),
                    np.concatenate([edge_vertices[:, 1], edge_vertices[:, 0]])])
    return {"h2v": h2v, "v2h": v2h, "v2e": v2e, "e2v": e2v, "v2v": v2v}


def _tables():
    ei = _topo()
    tabs = {}
    for (s, r, d) in _RELS:
        src, dst = ei[r]
        ns, nd = _COUNTS[s], _COUNTS[d]
        deg = np.bincount(dst, minlength=nd)
        D = int(deg.max())
        G = np.zeros((nd * D, ns), np.float32)
        msk = np.zeros((nd, D), np.float32)
        fill = np.zeros(nd, np.int32)
        for e in range(src.shape[0]):
            dd = dst[e]
            G[dd * D + fill[dd], src[e]] = 1.0
            msk[dd, fill[dd]] = 1.0
            fill[dd] += 1
        tabs[r] = (G, msk, D)
    return tabs


_TABS = _tables()


def _body(hx_ref, vx_ref, ex_ref, pf_ref, oh_ref,
          pwh_ref, pwv_ref, pwe_ref, pb_ref,
          sws_ref, swd_ref, mw_ref, lng_ref, lnb_ref, bs_ref,
          w1_ref, b1_ref, w2_ref, b2_ref,
          g0_ref, g1_ref, g2_ref, g3_ref, g4_ref, msk_ref, o_ref):
    grefs = [g0_ref, g1_ref, g2_ref, g3_ref, g4_ref]
    x = {}
    for nt, fref, wref, bi in (("hex", hx_ref, pwh_ref, 0),
                               ("vertex", vx_ref, pwv_ref, 1),
                               ("edge", ex_ref, pwe_ref, 2)):
        n = _COUNTS[nt]
        f = fref[...].reshape(n, _IN_DIMS[nt])
        x[nt] = jnp.dot(f, wref[...], preferred_element_type=jnp.float32) \
            + pb_ref[bi, :][None, :]

    for li in range(_NL):
        out = {nt: None for nt in _NTS}
        for ri, (st, r, dt) in enumerate(_RELS):
            ns, nd = _COUNTS[st], _COUNTS[dt]
            _, _, D = _TABS[r]
            msk = msk_ref[ri, :nd, :D]
            sl = jnp.dot(x[st], sws_ref[li, ri],
                         preferred_element_type=jnp.float32)
            sg = jnp.dot(grefs[ri][...], sl,
                         preferred_element_type=jnp.float32)
            sd = jnp.dot(x[dt], swd_ref[li, ri],
                         preferred_element_type=jnp.float32)
            a = sg.reshape(nd, D, _HEADS) + sd[:, None, :]
            a = jnp.where(a >= 0, a, 0.2 * a)
            gmax = jnp.max(jnp.max(a, axis=0), axis=0)
            e = jnp.exp(a - gmax[None, None, :]) * msk[:, :, None]
            den = jnp.sum(e, axis=1)
            recip = 0.25 / (den + 1e-16)
            alpha = (e * recip[:, None, :]).reshape(nd * D, _HEADS)
            msg = jnp.dot(x[st], mw_ref[li, ri],
                          preferred_element_type=jnp.float32)
            mg = jnp.dot(grefs[ri][...], msg,
                         preferred_element_type=jnp.float32)
            wm = None
            for h in range(_HEADS):
                t = mg[:, h * _HID:(h + 1) * _HID] * alpha[:, h:h + 1]
                wm = t if wm is None else wm + t
            contrib = jnp.sum(wm.reshape(nd, D, _HID), axis=1)
            out[dt] = contrib if out[dt] is None else out[dt] + contrib

        for ni, nt in enumerate(_NTS):
            z = out[nt] + bs_ref[li, ni][None, :]
            mu = jnp.mean(z, axis=-1, keepdims=True)
            zc = z - mu
            var = jnp.mean(zc * zc, axis=-1, keepdims=True)
            zn = zc * jax.lax.rsqrt(var + 1e-5)
            zn = zn * lng_ref[li, ni][None, :] + lnb_ref[li, ni][None, :]
            x[nt] = jnp.maximum(zn, 0.0)

    pools = [jnp.mean(x[nt], axis=0, keepdims=True) for nt in _NTS]
    pf = pf_ref[...].reshape(_N_PLAYERS, _PLAYER_DIM)
    oh = oh_ref[...].reshape(1, _N_PLAYERS)
    cur = jnp.dot(oh, pf, preferred_element_type=jnp.float32)
    h = jnp.concatenate(pools + [cur], axis=-1)
    z = jnp.dot(h, w1_ref[...], preferred_element_type=jnp.float32) \
        + b1_ref[...]
    z = jnp.maximum(z, 0.0)
    o = jnp.dot(z, w2_ref[...],
                preferred_element_type=jnp.float32) + b2_ref[...]
    o_ref[...] = o.reshape(1, 1, _OUT_DIM)


def kernel(hex_features, vertex_features, edge_features, player_features,
           current_player, params):
    bsz = hex_features.shape[0]

    oh = jax.nn.one_hot(current_player, _N_PLAYERS,
                        dtype=jnp.float32).reshape(bsz, 1, _N_PLAYERS)

    proj = params["proj"]
    pb = jnp.stack([proj[nt]["b"] for nt in _NTS])

    sws_l, swd_l, mw_l, lng_l, lnb_l, bs_l = [], [], [], [], [], []
    for layer in params["layers"]:
        convs = layer["convs"]
        def fold(r, key):
            w = convs[r]["w"].reshape(_HID, _HEADS, _HID)
            att = convs[r][key][0]
            return jnp.einsum("chk,hk->ch", w, att)
        sws_l.append(jnp.stack([fold(r, "att_src") for (_, r, _) in _RELS]))
        swd_l.append(jnp.stack([fold(r, "att_dst") for (_, r, _) in _RELS]))
        mw_l.append(jnp.stack([convs[r]["w"] for (_, r, _) in _RELS]))
        lng_l.append(jnp.stack([layer["norms"][nt]["g"] for nt in _NTS]))
        lnb_l.append(jnp.stack([layer["norms"][nt]["b"] for nt in _NTS]))
        bsum = []
        for nt in _NTS:
            b = jnp.zeros((_HID,), jnp.float32)
            for (st, r, dt) in _RELS:
                if dt == nt:
                    b = b + convs[r]["bias"]
            bsum.append(b)
        bs_l.append(jnp.stack(bsum))
    sws = jnp.stack(sws_l)
    swd = jnp.stack(swd_l)
    mw = jnp.stack(mw_l)
    lng = jnp.stack(lng_l)
    lnb = jnp.stack(lnb_l)
    bs = jnp.stack(bs_l)

    m = params["mlp"]
    b1 = m["b1"].reshape(1, _HID)
    b2 = m["b2"].reshape(1, _OUT_DIM)

    gmats = [jnp.asarray(_TABS[r][0]) for (_, r, _) in _RELS]
    msk_np = np.zeros((5, 72, 8), np.float32)
    for ri, (st, r, dt) in enumerate(_RELS):
        _, msk_t, D = _TABS[r]
        nd = _COUNTS[dt]
        msk_np[ri, :nd, :D] = msk_t
    msks = jnp.asarray(msk_np)

    full = lambda shp: pl.BlockSpec(shp, lambda i: tuple(0 for _ in shp))
    in_specs = [
        pl.BlockSpec((1, _COUNTS["hex"], _IN_DIMS["hex"]), lambda i: (i, 0, 0)),
        pl.BlockSpec((1, _COUNTS["vertex"], _IN_DIMS["vertex"]), lambda i: (i, 0, 0)),
        pl.BlockSpec((1, _COUNTS["edge"], _IN_DIMS["edge"]), lambda i: (i, 0, 0)),
        pl.BlockSpec((1, _N_PLAYERS, _PLAYER_DIM), lambda i: (i, 0, 0)),
        pl.BlockSpec((1, 1, _N_PLAYERS), lambda i: (i, 0, 0)),
        full(proj["hex"]["w"].shape),
        full(proj["vertex"]["w"].shape),
        full(proj["edge"]["w"].shape),
        full(pb.shape),
        full(sws.shape), full(swd.shape), full(mw.shape), full(lng.shape),
        full(lnb.shape),
        full(bs.shape),
        full(m["w1"].shape), full(b1.shape), full(m["w2"].shape),
        full(b2.shape),
    ] + [full(g.shape) for g in gmats] + [full(msks.shape)]
    out = pl.pallas_call(
        _body,
        grid=(bsz,),
        in_specs=in_specs,
        out_specs=pl.BlockSpec((1, 1, _OUT_DIM), lambda i: (i, 0, 0)),
        out_shape=jax.ShapeDtypeStruct((bsz, 1, _OUT_DIM), jnp.float32),
    )(hex_features, vertex_features, edge_features, player_features, oh,
      proj["hex"]["w"], proj["vertex"]["w"], proj["edge"]["w"], pb,
      sws, swd, mw, lng, lnb, bs, m["w1"], b1, m["w2"], b2,
      *gmats, msks)
    return out.reshape(bsz, _OUT_DIM)


# speedup vs baseline: 16.4702x; 1.0262x over previous
import numpy as np
import jax
import jax.numpy as jnp
from jax.experimental import pallas as pl

_GT = 8
_HID = 64
_HEADS = 4
_OUT_DIM = 256
_PLAYER_DIM = 14
_N_PLAYERS = 4
_NL = 3
_IN_DIMS = {"hex": 9, "vertex": 7, "edge": 5}
_COUNTS = {"hex": 19, "vertex": 54, "edge": 72}
_RELS = [("hex", "h2v", "vertex"), ("vertex", "v2h", "hex"),
         ("vertex", "v2e", "edge"), ("edge", "e2v", "vertex"),
         ("vertex", "v2v", "vertex")]
_NTS = ["hex", "vertex", "edge"]


def _topo():
    rng = np.random.RandomState(0)
    hex_vertices = rng.randint(0, 54, size=(19, 6))
    v0 = rng.randint(0, 54, size=72)
    v1 = (v0 + 1 + rng.randint(0, 53, size=72)) % 54
    edge_vertices = np.stack([v0, v1], axis=1)
    h2v = np.stack([np.repeat(np.arange(19), 6), hex_vertices.reshape(-1)])
    v2h = h2v[::-1].copy()
    e2v = np.stack([np.repeat(np.arange(72), 2), edge_vertices.reshape(-1)])
    v2e = e2v[::-1].copy()
    v2v = np.stack([np.concatenate([edge_vertices[:, 0], edge_vertices[:, 1]]),
                    np.concatenate([edge_vertices[:, 1], edge_vertices[:, 0]])])
    return {"h2v": h2v, "v2h": v2h, "v2e": v2e, "e2v": e2v, "v2v": v2v}


def _tables():
    ei = _topo()
    tabs = {}
    for (s, r, d) in _RELS:
        src, dst = ei[r]
        ns, nd = _COUNTS[s], _COUNTS[d]
        deg = np.bincount(dst, minlength=nd)
        D = int(deg.max())
        G = np.zeros((nd * D, ns), np.float32)
        msk = np.zeros((nd, D), np.float32)
        fill = np.zeros(nd, np.int32)
        for e in range(src.shape[0]):
            dd = dst[e]
            G[dd * D + fill[dd], src[e]] = 1.0
            msk[dd, fill[dd]] = 1.0
            fill[dd] += 1
        tabs[r] = (G, msk, D)
    return tabs


_TABS = _tables()


def _body(hx_ref, vx_ref, ex_ref, pf_ref, oh_ref,
          pwh_ref, pwv_ref, pwe_ref, pb_ref,
          sws_ref, swd_ref, mw_ref, lng_ref, lnb_ref, bs_ref,
          w1_ref, b1_ref, w2_ref, b2_ref,
          g0_ref, g1_ref, g2_ref, g3_ref, g4_ref, msk_ref, o_ref):
    grefs = [g0_ref, g1_ref, g2_ref, g3_ref, g4_ref]
    gt = _GT
    x = {}
    for nt, fref, wref, bi in (("hex", hx_ref, pwh_ref, 0),
                               ("vertex", vx_ref, pwv_ref, 1),
                               ("edge", ex_ref, pwe_ref, 2)):
        n = _COUNTS[nt]
        f = fref[...].reshape(gt * n, _IN_DIMS[nt])
        x[nt] = jnp.dot(f, wref[...], preferred_element_type=jnp.float32) \
            + pb_ref[bi, :][None, :]

    for li in range(_NL):
        out = {nt: None for nt in _NTS}
        for ri, (st, r, dt) in enumerate(_RELS):
            ns, nd = _COUNTS[st], _COUNTS[dt]
            _, _, D = _TABS[r]
            msk = msk_ref[ri, :nd, :D]
            G = grefs[ri][...]
            sl = jnp.dot(x[st], sws_ref[li, ri],
                         preferred_element_type=jnp.float32)
            sd = jnp.dot(x[dt], swd_ref[li, ri],
                         preferred_element_type=jnp.float32)
            msg = jnp.dot(x[st], mw_ref[li, ri],
                          preferred_element_type=jnp.float32)
            sg = jnp.concatenate(
                [jnp.dot(G, sl[g * ns:(g + 1) * ns],
                         preferred_element_type=jnp.float32)
                 for g in range(gt)], axis=0)
            mg = jnp.concatenate(
                [jnp.dot(G, msg[g * ns:(g + 1) * ns],
                         preferred_element_type=jnp.float32)
                 for g in range(gt)], axis=0)
            a = sg.reshape(gt, nd, D, _HEADS) + sd.reshape(gt, nd, 1, _HEADS)
            a = jnp.where(a >= 0, a, 0.2 * a)
            gmax = jnp.max(jnp.max(a, axis=2), axis=1)
            e = jnp.exp(a - gmax[:, None, None, :]) * msk[None, :, :, None]
            den = jnp.sum(e, axis=2)
            recip = 0.25 / (den + 1e-16)
            alpha = (e * recip[:, :, None, :]).reshape(gt * nd * D, _HEADS)
            wm = None
            for h in range(_HEADS):
                t = mg[:, h * _HID:(h + 1) * _HID] * alpha[:, h:h + 1]
                wm = t if wm is None else wm + t
            contrib = jnp.sum(wm.reshape(gt * nd, D, _HID), axis=1)
            out[dt] = contrib if out[dt] is None else out[dt] + contrib

        for ni, nt in enumerate(_NTS):
            z = out[nt] + bs_ref[li, ni][None, :]
            mu = jnp.mean(z, axis=-1, keepdims=True)
            zc = z - mu
            var = jnp.mean(zc * zc, axis=-1, keepdims=True)
            zn = zc * jax.lax.rsqrt(var + 1e-5)
            zn = zn * lng_ref[li, ni][None, :] + lnb_ref[li, ni][None, :]
            x[nt] = jnp.maximum(zn, 0.0)

    pools = [jnp.mean(x[nt].reshape(gt, _COUNTS[nt], _HID), axis=1)
             for nt in _NTS]
    pf = pf_ref[...]
    oh = oh_ref[...].reshape(gt, _N_PLAYERS)
    cur = jnp.sum(pf * oh[:, :, None], axis=1)
    h = jnp.concatenate(pools + [cur], axis=-1)
    z = jnp.dot(h, w1_ref[...], preferred_element_type=jnp.float32) \
        + b1_ref[...]
    z = jnp.maximum(z, 0.0)
    o = jnp.dot(z, w2_ref[...],
                preferred_element_type=jnp.float32) + b2_ref[...]
    o_ref[...] = o.reshape(gt, 1, _OUT_DIM)


def kernel(hex_features, vertex_features, edge_features, player_features,
           current_player, params):
    bsz = hex_features.shape[0]

    oh = jax.nn.one_hot(current_player, _N_PLAYERS,
                        dtype=jnp.float32).reshape(bsz, 1, _N_PLAYERS)

    proj = params["proj"]
    pb = jnp.stack([proj[nt]["b"] for nt in _NTS])

    sws_l, swd_l, mw_l, lng_l, lnb_l, bs_l = [], [], [], [], [], []
    for layer in params["layers"]:
        convs = layer["convs"]
        def fold(r, key):
            w = convs[r]["w"].reshape(_HID, _HEADS, _HID)
            att = convs[r][key][0]
            return jnp.einsum("chk,hk->ch", w, att)
        sws_l.append(jnp.stack([fold(r, "att_src") for (_, r, _) in _RELS]))
        swd_l.append(jnp.stack([fold(r, "att_dst") for (_, r, _) in _RELS]))
        mw_l.append(jnp.stack([convs[r]["w"] for (_, r, _) in _RELS]))
        lng_l.append(jnp.stack([layer["norms"][nt]["g"] for nt in _NTS]))
        lnb_l.append(jnp.stack([layer["norms"][nt]["b"] for nt in _NTS]))
        bsum = []
        for nt in _NTS:
            b = jnp.zeros((_HID,), jnp.float32)
            for (st, r, dt) in _RELS:
                if dt == nt:
                    b = b + convs[r]["bias"]
            bsum.append(b)
        bs_l.append(jnp.stack(bsum))
    sws = jnp.stack(sws_l)
    swd = jnp.stack(swd_l)
    mw = jnp.stack(mw_l)
    lng = jnp.stack(lng_l)
    lnb = jnp.stack(lnb_l)
    bs = jnp.stack(bs_l)

    m = params["mlp"]
    b1 = m["b1"].reshape(1, _HID)
    b2 = m["b2"].reshape(1, _OUT_DIM)

    gmats = [jnp.asarray(_TABS[r][0]) for (_, r, _) in _RELS]
    msk_np = np.zeros((5, 72, 8), np.float32)
    for ri, (st, r, dt) in enumerate(_RELS):
        _, msk_t, D = _TABS[r]
        nd = _COUNTS[dt]
        msk_np[ri, :nd, :D] = msk_t
    msks = jnp.asarray(msk_np)

    full = lambda shp: pl.BlockSpec(shp, lambda i: tuple(0 for _ in shp))
    in_specs = [
        pl.BlockSpec((_GT, _COUNTS["hex"], _IN_DIMS["hex"]), lambda i: (i, 0, 0)),
        pl.BlockSpec((_GT, _COUNTS["vertex"], _IN_DIMS["vertex"]), lambda i: (i, 0, 0)),
        pl.BlockSpec((_GT, _COUNTS["edge"], _IN_DIMS["edge"]), lambda i: (i, 0, 0)),
        pl.BlockSpec((_GT, _N_PLAYERS, _PLAYER_DIM), lambda i: (i, 0, 0)),
        pl.BlockSpec((_GT, 1, _N_PLAYERS), lambda i: (i, 0, 0)),
        full(proj["hex"]["w"].shape),
        full(proj["vertex"]["w"].shape),
        full(proj["edge"]["w"].shape),
        full(pb.shape),
        full(sws.shape), full(swd.shape), full(mw.shape), full(lng.shape),
        full(lnb.shape),
        full(bs.shape),
        full(m["w1"].shape), full(b1.shape), full(m["w2"].shape),
        full(b2.shape),
    ] + [full(g.shape) for g in gmats] + [full(msks.shape)]
    out = pl.pallas_call(
        _body,
        grid=(bsz // _GT,),
        in_specs=in_specs,
        out_specs=pl.BlockSpec((_GT, 1, _OUT_DIM), lambda i: (i, 0, 0)),
        out_shape=jax.ShapeDtypeStruct((bsz, 1, _OUT_DIM), jnp.float32),
    )(hex_features, vertex_features, edge_features, player_features, oh,
      proj["hex"]["w"], proj["vertex"]["w"], proj["edge"]["w"], pb,
      sws, swd, mw, lng, lnb, bs, m["w1"], b1, m["w2"], b2,
      *gmats, msks)
    return out.reshape(bsz, _OUT_DIM)


# speedup vs baseline: 16.5337x; 1.0039x over previous
import numpy as np
import jax
import jax.numpy as jnp
from jax.experimental import pallas as pl

_GT = 8
_HID = 64
_HEADS = 4
_OUT_DIM = 256
_PLAYER_DIM = 14
_N_PLAYERS = 4
_NL = 3
_IN_DIMS = {"hex": 9, "vertex": 7, "edge": 5}
_COUNTS = {"hex": 19, "vertex": 54, "edge": 72}
_RELS = [("hex", "h2v", "vertex"), ("vertex", "v2h", "hex"),
         ("vertex", "v2e", "edge"), ("edge", "e2v", "vertex"),
         ("vertex", "v2v", "vertex")]
_NTS = ["hex", "vertex", "edge"]


def _topo():
    rng = np.random.RandomState(0)
    hex_vertices = rng.randint(0, 54, size=(19, 6))
    v0 = rng.randint(0, 54, size=72)
    v1 = (v0 + 1 + rng.randint(0, 53, size=72)) % 54
    edge_vertices = np.stack([v0, v1], axis=1)
    h2v = np.stack([np.repeat(np.arange(19), 6), hex_vertices.reshape(-1)])
    v2h = h2v[::-1].copy()
    e2v = np.stack([np.repeat(np.arange(72), 2), edge_vertices.reshape(-1)])
    v2e = e2v[::-1].copy()
    v2v = np.stack([np.concatenate([edge_vertices[:, 0], edge_vertices[:, 1]]),
                    np.concatenate([edge_vertices[:, 1], edge_vertices[:, 0]])])
    return {"h2v": h2v, "v2h": v2h, "v2e": v2e, "e2v": e2v, "v2v": v2v}


def _tables():
    ei = _topo()
    tabs = {}
    for (s, r, d) in _RELS:
        src, dst = ei[r]
        ns, nd = _COUNTS[s], _COUNTS[d]
        deg = np.bincount(dst, minlength=nd)
        D = int(deg.max())
        G = np.zeros((nd * D, ns), np.float32)
        msk = np.zeros((nd, D), np.float32)
        fill = np.zeros(nd, np.int32)
        for e in range(src.shape[0]):
            dd = dst[e]
            G[dd * D + fill[dd], src[e]] = 1.0
            msk[dd, fill[dd]] = 1.0
            fill[dd] += 1
        tabs[r] = (G, msk, D)
    return tabs


_TABS = _tables()


def _body(hx_ref, vx_ref, ex_ref, pf_ref, oh_ref,
          pwh_ref, pwv_ref, pwe_ref, pb_ref,
          sws_ref, swd_ref, mw_ref, lng_ref, lnb_ref, bs_ref,
          w1_ref, b1_ref, w2_ref, b2_ref,
          g0_ref, g1_ref, g2_ref, g3_ref, g4_ref, msk_ref, o_ref):
    grefs = [g0_ref, g1_ref, g2_ref, g3_ref, g4_ref]
    gt = _GT
    x = {}
    for nt, fref, wref, bi in (("hex", hx_ref, pwh_ref, 0),
                               ("vertex", vx_ref, pwv_ref, 1),
                               ("edge", ex_ref, pwe_ref, 2)):
        n = _COUNTS[nt]
        f = fref[...].reshape(gt * n, _IN_DIMS[nt])
        x[nt] = jnp.dot(f, wref[...], preferred_element_type=jnp.float32) \
            + pb_ref[bi, :][None, :]

    for li in range(_NL):
        out = {nt: None for nt in _NTS}
        for ri, (st, r, dt) in enumerate(_RELS):
            ns, nd = _COUNTS[st], _COUNTS[dt]
            _, _, D = _TABS[r]
            msk = msk_ref[ri, :nd, :D]
            G = grefs[ri][...]
            sl = jnp.dot(x[st], sws_ref[li, ri],
                         preferred_element_type=jnp.float32)
            sd = jnp.dot(x[dt], swd_ref[li, ri],
                         preferred_element_type=jnp.float32)
            msg = jnp.dot(x[st], mw_ref[li, ri],
                          preferred_element_type=jnp.float32)
            ml = jnp.concatenate([msg, sl], axis=1)
            mlg = jnp.concatenate(
                [jnp.dot(G, ml[g * ns:(g + 1) * ns],
                         preferred_element_type=jnp.float32)
                 for g in range(gt)], axis=0)
            mg = mlg[:, :_HEADS * _HID]
            sg = mlg[:, _HEADS * _HID:]
            a = sg.reshape(gt, nd, D, _HEADS) + sd.reshape(gt, nd, 1, _HEADS)
            a = jnp.where(a >= 0, a, 0.2 * a)
            gmax = jnp.max(jnp.max(a, axis=2), axis=1)
            e = jnp.exp(a - gmax[:, None, None, :]) * msk[None, :, :, None]
            den = jnp.sum(e, axis=2)
            recip = 0.25 / (den + 1e-16)
            alpha = (e * recip[:, :, None, :]).reshape(gt * nd * D, _HEADS)
            wm = None
            for h in range(_HEADS):
                t = mg[:, h * _HID:(h + 1) * _HID] * alpha[:, h:h + 1]
                wm = t if wm is None else wm + t
            contrib = jnp.sum(wm.reshape(gt * nd, D, _HID), axis=1)
            out[dt] = contrib if out[dt] is None else out[dt] + contrib

        for ni, nt in enumerate(_NTS):
            z = out[nt] + bs_ref[li, ni][None, :]
            mu = jnp.mean(z, axis=-1, keepdims=True)
            zc = z - mu
            var = jnp.mean(zc * zc, axis=-1, keepdims=True)
            zn = zc * jax.lax.rsqrt(var + 1e-5)
            zn = zn * lng_ref[li, ni][None, :] + lnb_ref[li, ni][None, :]
            x[nt] = jnp.maximum(zn, 0.0)

    pools = [jnp.mean(x[nt].reshape(gt, _COUNTS[nt], _HID), axis=1)
             for nt in _NTS]
    pf = pf_ref[...]
    oh = oh_ref[...].reshape(gt, _N_PLAYERS)
    cur = jnp.sum(pf * oh[:, :, None], axis=1)
    h = jnp.concatenate(pools + [cur], axis=-1)
    z = jnp.dot(h, w1_ref[...], preferred_element_type=jnp.float32) \
        + b1_ref[...]
    z = jnp.maximum(z, 0.0)
    o = jnp.dot(z, w2_ref[...],
                preferred_element_type=jnp.float32) + b2_ref[...]
    o_ref[...] = o.reshape(gt, 1, _OUT_DIM)


def kernel(hex_features, vertex_features, edge_features, player_features,
           current_player, params):
    bsz = hex_features.shape[0]

    oh = jax.nn.one_hot(current_player, _N_PLAYERS,
                        dtype=jnp.float32).reshape(bsz, 1, _N_PLAYERS)

    proj = params["proj"]
    pb = jnp.stack([proj[nt]["b"] for nt in _NTS])

    sws_l, swd_l, mw_l, lng_l, lnb_l, bs_l = [], [], [], [], [], []
    for layer in params["layers"]:
        convs = layer["convs"]
        def fold(r, key):
            w = convs[r]["w"].reshape(_HID, _HEADS, _HID)
            att = convs[r][key][0]
            return jnp.einsum("chk,hk->ch", w, att)
        sws_l.append(jnp.stack([fold(r, "att_src") for (_, r, _) in _RELS]))
        swd_l.append(jnp.stack([fold(r, "att_dst") for (_, r, _) in _RELS]))
        mw_l.append(jnp.stack([convs[r]["w"] for (_, r, _) in _RELS]))
        lng_l.append(jnp.stack([layer["norms"][nt]["g"] for nt in _NTS]))
        lnb_l.append(jnp.stack([layer["norms"][nt]["b"] for nt in _NTS]))
        bsum = []
        for nt in _NTS:
            b = jnp.zeros((_HID,), jnp.float32)
            for (st, r, dt) in _RELS:
                if dt == nt:
                    b = b + convs[r]["bias"]
            bsum.append(b)
        bs_l.append(jnp.stack(bsum))
    sws = jnp.stack(sws_l)
    swd = jnp.stack(swd_l)
    mw = jnp.stack(mw_l)
    lng = jnp.stack(lng_l)
    lnb = jnp.stack(lnb_l)
    bs = jnp.stack(bs_l)

    m = params["mlp"]
    b1 = m["b1"].reshape(1, _HID)
    b2 = m["b2"].reshape(1, _OUT_DIM)

    gmats = [jnp.asarray(_TABS[r][0]) for (_, r, _) in _RELS]
    msk_np = np.zeros((5, 72, 8), np.float32)
    for ri, (st, r, dt) in enumerate(_RELS):
        _, msk_t, D = _TABS[r]
        nd = _COUNTS[dt]
        msk_np[ri, :nd, :D] = msk_t
    msks = jnp.asarray(msk_np)

    full = lambda shp: pl.BlockSpec(shp, lambda i: tuple(0 for _ in shp))
    in_specs = [
        pl.BlockSpec((_GT, _COUNTS["hex"], _IN_DIMS["hex"]), lambda i: (i, 0, 0)),
        pl.BlockSpec((_GT, _COUNTS["vertex"], _IN_DIMS["vertex"]), lambda i: (i, 0, 0)),
        pl.BlockSpec((_GT, _COUNTS["edge"], _IN_DIMS["edge"]), lambda i: (i, 0, 0)),
        pl.BlockSpec((_GT, _N_PLAYERS, _PLAYER_DIM), lambda i: (i, 0, 0)),
        pl.BlockSpec((_GT, 1, _N_PLAYERS), lambda i: (i, 0, 0)),
        full(proj["hex"]["w"].shape),
        full(proj["vertex"]["w"].shape),
        full(proj["edge"]["w"].shape),
        full(pb.shape),
        full(sws.shape), full(swd.shape), full(mw.shape), full(lng.shape),
        full(lnb.shape),
        full(bs.shape),
        full(m["w1"].shape), full(b1.shape), full(m["w2"].shape),
        full(b2.shape),
    ] + [full(g.shape) for g in gmats] + [full(msks.shape)]
    out = pl.pallas_call(
        _body,
        grid=(bsz // _GT,),
        in_specs=in_specs,
        out_specs=pl.BlockSpec((_GT, 1, _OUT_DIM), lambda i: (i, 0, 0)),
        out_shape=jax.ShapeDtypeStruct((bsz, 1, _OUT_DIM), jnp.float32),
    )(hex_features, vertex_features, edge_features, player_features, oh,
      proj["hex"]["w"], proj["vertex"]["w"], proj["edge"]["w"], pb,
      sws, swd, mw, lng, lnb, bs, m["w1"], b1, m["w2"], b2,
      *gmats, msks)
    return out.reshape(bsz, _OUT_DIM)
